# Initial kernel scaffold; baseline (speedup 1.0000x reference)
import jax
import jax.numpy as jnp
from jax.experimental import pallas as pl


def kernel(h_src, h_dst, attn_l, edge_index):
    raise NotImplementedError("write your pallas kernel here")


# speedup vs baseline: 93.8634x; 93.8634x over previous
import functools

import jax
import jax.numpy as jnp
from jax import lax
from jax.experimental import pallas as pl
from jax.experimental.pallas import tpu as pltpu
from jax.experimental.pallas import tpu_sc as plsc

_N = 10000
_E = 320000
_H = 8
_D = 16
_NEG_SLOPE = 0.2
_HD = _H * _D
_W = 144

_NC = 2
_NS = 16
_NW = _NC * _NS
_EPT = _E // _NW
_C = 80
_ITERS = _EPT // _C
_RPT8 = 624
_TAIL = _N - _NS * _RPT8


def _prep_body(hs_ref, attn_ref, gsel_ref, gselt_ref, out_ref):
    hs = hs_ref[...]
    attn = attn_ref[...]
    gsel = gsel_ref[...]
    gselt = gselt_ref[...]
    el = jnp.dot(hs * attn, gsel, precision=lax.Precision.HIGHEST)
    el = jnp.where(el > 0, el, _NEG_SLOPE * el)
    m = jnp.max(el)
    q = jnp.exp(el - m)
    qb = jnp.dot(q, gselt, precision=lax.Precision.HIGHEST)
    w = hs * qb
    pad = jnp.zeros((hs.shape[0], _W - _HD - _H), jnp.float32)
    out_ref[...] = jnp.concatenate([w, q, pad], axis=1)


def _prep(h_src, attn_flat, gsel, gselt):
    return pl.pallas_call(
        _prep_body,
        out_shape=jax.ShapeDtypeStruct((_N, _W), jnp.float32),
    )(h_src, attn_flat, gsel, gselt)


def _edge_body(w_hbm, src_hbm, dst_hbm, zeros_hbm, out_hbm,
               sidx, didx, rows, acc, sem):
    c = lax.axis_index("c")
    s = lax.axis_index("s")
    wid = s * _NC + c

    zoff = pl.multiple_of(s * _RPT8, 8)
    pltpu.sync_copy(zeros_hbm.at[pl.ds(zoff, _RPT8)],
                    acc.at[pl.ds(zoff, _RPT8)])

    @pl.when(s == _NS - 1)
    def _zero_tail():
        pltpu.sync_copy(zeros_hbm.at[pl.ds(_NS * _RPT8, _TAIL)],
                        acc.at[pl.ds(_NS * _RPT8, _TAIL)])

    plsc.subcore_barrier()

    base = wid * _EPT

    def body(i, carry):
        off = pl.multiple_of(base + i * _C, 8)
        pltpu.sync_copy(src_hbm.at[pl.ds(off, _C)], sidx)
        pltpu.sync_copy(dst_hbm.at[pl.ds(off, _C)], didx)
        pltpu.async_copy(w_hbm.at[sidx], rows, sem).wait()
        pltpu.sync_copy(rows, acc.at[didx], add=True)
        return carry

    lax.fori_loop(0, _ITERS, body, 0)
    plsc.subcore_barrier()

    pltpu.sync_copy(acc.at[pl.ds(zoff, _RPT8)],
                    out_hbm.at[c, pl.ds(zoff, _RPT8)])

    @pl.when(s == _NS - 1)
    def _drain_tail():
        pltpu.sync_copy(acc.at[pl.ds(_NS * _RPT8, _TAIL)],
                        out_hbm.at[c, pl.ds(_NS * _RPT8, _TAIL)])


@functools.partial(
    pl.kernel,
    mesh=plsc.VectorSubcoreMesh(core_axis_name="c", subcore_axis_name="s"),
    out_type=jax.ShapeDtypeStruct((_NC, _N, _W), jnp.float32),
    scratch_types=[
        pltpu.VMEM((_C,), jnp.int32),
        pltpu.VMEM((_C,), jnp.int32),
        pltpu.VMEM((_C, _W), jnp.float32),
        pltpu.VMEM_SHARED((_N, _W), jnp.float32),
        pltpu.SemaphoreType.DMA,
    ],
    compiler_params=pltpu.CompilerParams(use_tc_tiling_on_sc=False),
)
def _edge_pass(w_hbm, src_hbm, dst_hbm, zeros_hbm, out_hbm,
               sidx, didx, rows, acc, sem):
    _edge_body(w_hbm, src_hbm, dst_hbm, zeros_hbm, out_hbm,
               sidx, didx, rows, acc, sem)


def _final_body(acc_ref, gselt_ref, out_ref):
    tot = acc_ref[0] + acc_ref[1]
    num = tot[:, 0:_HD]
    den = tot[:, _HD:_HD + _H]
    denb = jnp.dot(den, gselt_ref[...], precision=lax.Precision.HIGHEST)
    out_ref[...] = jnp.where(denb > 0, num / denb, 0.0)


def _final(acc, gselt):
    return pl.pallas_call(
        _final_body,
        out_shape=jax.ShapeDtypeStruct((_N, _HD), jnp.float32),
    )(acc, gselt)


def kernel(h_src, h_dst, attn_l, edge_index):
    del h_dst
    attn_flat = attn_l.reshape(1, _HD).astype(jnp.float32)
    gsel = (jnp.arange(_HD)[:, None] // _D == jnp.arange(_H)[None, :]
            ).astype(jnp.float32)
    gselt = gsel.T
    w = _prep(h_src, attn_flat, gsel, gselt)
    src = edge_index[0]
    dst = edge_index[1]
    zeros = jnp.zeros((_N, _W), jnp.float32)
    acc = _edge_pass(w, src, dst, zeros)
    return _final(acc, gselt)


# speedup vs baseline: 161.8003x; 1.7238x over previous
import functools

import jax
import jax.numpy as jnp
from jax import lax
from jax.experimental import pallas as pl
from jax.experimental.pallas import tpu as pltpu
from jax.experimental.pallas import tpu_sc as plsc

_N = 10000
_E = 320000
_H = 8
_D = 16
_NEG_SLOPE = 0.2
_HD = _H * _D
_W = 144

_NC = 2
_NS = 16
_NW = _NC * _NS
_EPT = _E // _NW
_C = 125
_ITERS = _EPT // _C
_RPT8 = 624
_TAIL = _N - _NS * _RPT8


def _prep_body(hs_ref, attn_ref, gsel_ref, gselt_ref, out_ref):
    hs = hs_ref[...]
    attn = attn_ref[...]
    gsel = gsel_ref[...]
    gselt = gselt_ref[...]
    el = jnp.dot(hs * attn, gsel, precision=lax.Precision.HIGHEST)
    el = jnp.where(el > 0, el, _NEG_SLOPE * el)
    m = jnp.max(el)
    q = jnp.exp(el - m)
    qb = jnp.dot(q, gselt, precision=lax.Precision.HIGHEST)
    w = hs * qb
    pad = jnp.zeros((hs.shape[0], _W - _HD - _H), jnp.float32)
    out_ref[...] = jnp.concatenate([w, q, pad], axis=1)


def _prep(h_src, attn_flat, gsel, gselt):
    return pl.pallas_call(
        _prep_body,
        out_shape=jax.ShapeDtypeStruct((_N, _W), jnp.float32),
    )(h_src, attn_flat, gsel, gselt)


def _edge_body(w_hbm, src_hbm, dst_hbm, zeros_hbm, out_hbm,
               sidx0, sidx1, didx0, didx1, rows0, rows1, acc,
               gsem0, gsem1, isem0, isem1):
    c = lax.axis_index("c")
    s = lax.axis_index("s")
    wid = s * _NC + c
    cbase = wid * _ITERS

    zoff = pl.multiple_of(s * _RPT8, 8)
    pltpu.sync_copy(zeros_hbm.at[pl.ds(zoff, _RPT8)],
                    acc.at[pl.ds(zoff, _RPT8)])

    @pl.when(s == _NS - 1)
    def _zero_tail():
        pltpu.sync_copy(zeros_hbm.at[pl.ds(_NS * _RPT8, _TAIL)],
                        acc.at[pl.ds(_NS * _RPT8, _TAIL)])

    plsc.subcore_barrier()

    pltpu.async_copy(src_hbm.at[cbase], sidx0, isem0)
    pltpu.async_copy(dst_hbm.at[cbase], didx0, isem0)
    pltpu.async_copy(src_hbm.at[cbase + 1], sidx1, isem1)
    pltpu.async_copy(dst_hbm.at[cbase + 1], didx1, isem1)
    pltpu.make_async_copy(src_hbm.at[cbase], sidx0, isem0).wait()
    pltpu.make_async_copy(dst_hbm.at[cbase], didx0, isem0).wait()
    pltpu.async_copy(w_hbm.at[sidx0], rows0, gsem0)

    @pl.loop(0, _ITERS, step=2)
    def _pair(g):
        pltpu.make_async_copy(w_hbm.at[sidx0], rows0, gsem0).wait()
        pltpu.make_async_copy(src_hbm.at[cbase], sidx1, isem1).wait()
        pltpu.make_async_copy(dst_hbm.at[cbase], didx1, isem1).wait()
        pltpu.async_copy(w_hbm.at[sidx1], rows1, gsem1)
        pltpu.sync_copy(rows0, acc.at[didx0], add=True)

        @pl.when(g + 2 < _ITERS)
        def _idx2():
            pltpu.async_copy(src_hbm.at[cbase + g + 2], sidx0, isem0)
            pltpu.async_copy(dst_hbm.at[cbase + g + 2], didx0, isem0)

        pltpu.make_async_copy(w_hbm.at[sidx1], rows1, gsem1).wait()

        @pl.when(g + 2 < _ITERS)
        def _g2():
            pltpu.make_async_copy(src_hbm.at[cbase], sidx0, isem0).wait()
            pltpu.make_async_copy(dst_hbm.at[cbase], didx0, isem0).wait()
            pltpu.async_copy(w_hbm.at[sidx0], rows0, gsem0)

        pltpu.sync_copy(rows1, acc.at[didx1], add=True)

        @pl.when(g + 3 < _ITERS)
        def _idx3():
            pltpu.async_copy(src_hbm.at[cbase + g + 3], sidx1, isem1)
            pltpu.async_copy(dst_hbm.at[cbase + g + 3], didx1, isem1)

    plsc.subcore_barrier()

    pltpu.sync_copy(acc.at[pl.ds(zoff, _RPT8)],
                    out_hbm.at[c, pl.ds(zoff, _RPT8)])

    @pl.when(s == _NS - 1)
    def _drain_tail():
        pltpu.sync_copy(acc.at[pl.ds(_NS * _RPT8, _TAIL)],
                        out_hbm.at[c, pl.ds(_NS * _RPT8, _TAIL)])


@functools.partial(
    pl.kernel,
    mesh=plsc.VectorSubcoreMesh(core_axis_name="c", subcore_axis_name="s"),
    out_type=jax.ShapeDtypeStruct((_NC, _N, _W), jnp.float32),
    scratch_types=[
        pltpu.VMEM((_C,), jnp.int32),
        pltpu.VMEM((_C,), jnp.int32),
        pltpu.VMEM((_C,), jnp.int32),
        pltpu.VMEM((_C,), jnp.int32),
        pltpu.VMEM((_C, _W), jnp.float32),
        pltpu.VMEM((_C, _W), jnp.float32),
        pltpu.VMEM_SHARED((_N, _W), jnp.float32),
        pltpu.SemaphoreType.DMA,
        pltpu.SemaphoreType.DMA,
        pltpu.SemaphoreType.DMA,
        pltpu.SemaphoreType.DMA,
    ],
    compiler_params=pltpu.CompilerParams(use_tc_tiling_on_sc=False),
)
def _edge_pass(w_hbm, src_hbm, dst_hbm, zeros_hbm, out_hbm,
               sidx0, sidx1, didx0, didx1, rows0, rows1, acc,
               gsem0, gsem1, isem0, isem1):
    _edge_body(w_hbm, src_hbm, dst_hbm, zeros_hbm, out_hbm,
               sidx0, sidx1, didx0, didx1, rows0, rows1, acc,
               gsem0, gsem1, isem0, isem1)


def _final_body(acc_ref, gselt_ref, out_ref):
    tot = acc_ref[0] + acc_ref[1]
    num = tot[:, 0:_HD]
    den = tot[:, _HD:_HD + _H]
    denb = jnp.dot(den, gselt_ref[...], precision=lax.Precision.HIGHEST)
    out_ref[...] = jnp.where(denb > 0, num / denb, 0.0)


def _final(acc, gselt):
    return pl.pallas_call(
        _final_body,
        out_shape=jax.ShapeDtypeStruct((_N, _HD), jnp.float32),
    )(acc, gselt)


def kernel(h_src, h_dst, attn_l, edge_index):
    del h_dst
    attn_flat = attn_l.reshape(1, _HD).astype(jnp.float32)
    gsel = (jnp.arange(_HD)[:, None] // _D == jnp.arange(_H)[None, :]
            ).astype(jnp.float32)
    gselt = gsel.T
    w = _prep(h_src, attn_flat, gsel, gselt)
    src = edge_index[0].reshape(-1, _C)
    dst = edge_index[1].reshape(-1, _C)
    zeros = jnp.zeros((_N, _W), jnp.float32)
    acc = _edge_pass(w, src, dst, zeros)
    return _final(acc, gselt)


# speedup vs baseline: 173.7718x; 1.0740x over previous
import functools

import jax
import jax.numpy as jnp
from jax import lax
from jax.experimental import pallas as pl
from jax.experimental.pallas import tpu as pltpu
from jax.experimental.pallas import tpu_sc as plsc

_N = 10000
_E = 320000
_H = 8
_D = 16
_NEG_SLOPE = 0.2
_HD = _H * _D
_W = 144

_NC = 2
_NS = 16
_NW = _NC * _NS
_EPT = _E // _NW
_C = 125
_ITERS = _EPT // _C
_RPT8 = 624
_TAIL = _N - _NS * _RPT8


def _prep_body(hs_ref, attn_ref, gsel_ref, gselt_ref, out_ref):
    hs = hs_ref[...]
    attn = attn_ref[...]
    gsel = gsel_ref[...]
    gselt = gselt_ref[...]
    el = jnp.dot(hs * attn, gsel, precision=lax.Precision.HIGHEST)
    el = jnp.where(el > 0, el, _NEG_SLOPE * el)
    m = jnp.max(el)
    q = jnp.exp(el - m)
    qb = jnp.dot(q, gselt, precision=lax.Precision.HIGHEST)
    w = hs * qb
    pad = jnp.zeros((hs.shape[0], _W - _HD - _H), jnp.float32)
    out_ref[...] = jnp.concatenate([w, q, pad], axis=1)


def _prep(h_src, attn_flat, gsel, gselt):
    return pl.pallas_call(
        _prep_body,
        out_shape=jax.ShapeDtypeStruct((_N, _W), jnp.float32),
    )(h_src, attn_flat, gsel, gselt)


def _edge_body(w_hbm, ei_hbm, out_hbm,
               sidx0, sidx1, didx0, didx1, rows0, rows1, acc,
               gsem0, gsem1, isem0, isem1):
    c = lax.axis_index("c")
    s = lax.axis_index("s")
    wid = s * _NC + c
    sbase = wid * _ITERS
    dbase = _E // _C + wid * _ITERS

    @pl.loop(0, _C)
    def _zrow(r):
        for k in range(_W // 16):
            rows0[r, pl.ds(16 * k, 16)] = jnp.zeros((16,), jnp.float32)

    @pl.loop(0, 6)
    def _zcp(j):
        zoff = pl.multiple_of(s * _RPT8 + j * 104, 8)
        pltpu.sync_copy(rows0.at[pl.ds(0, 104)], acc.at[pl.ds(zoff, 104)])

    @pl.when(s == _NS - 1)
    def _zero_tail():
        pltpu.sync_copy(rows0.at[pl.ds(0, _TAIL)],
                        acc.at[pl.ds(_NS * _RPT8, _TAIL)])

    plsc.subcore_barrier()

    pltpu.async_copy(ei_hbm.at[sbase], sidx0, isem0)
    pltpu.async_copy(ei_hbm.at[dbase], didx0, isem0)
    pltpu.async_copy(ei_hbm.at[sbase + 1], sidx1, isem1)
    pltpu.async_copy(ei_hbm.at[dbase + 1], didx1, isem1)
    pltpu.make_async_copy(ei_hbm.at[sbase], sidx0, isem0).wait()
    pltpu.make_async_copy(ei_hbm.at[dbase], didx0, isem0).wait()
    pltpu.async_copy(w_hbm.at[sidx0], rows0, gsem0)

    @pl.loop(0, _ITERS, step=2)
    def _pair(g):
        pltpu.make_async_copy(w_hbm.at[sidx0], rows0, gsem0).wait()
        pltpu.make_async_copy(ei_hbm.at[sbase], sidx1, isem1).wait()
        pltpu.make_async_copy(ei_hbm.at[dbase], didx1, isem1).wait()
        pltpu.async_copy(w_hbm.at[sidx1], rows1, gsem1)
        pltpu.sync_copy(rows0, acc.at[didx0], add=True)

        @pl.when(g + 2 < _ITERS)
        def _idx2():
            pltpu.async_copy(ei_hbm.at[sbase + g + 2], sidx0, isem0)
            pltpu.async_copy(ei_hbm.at[dbase + g + 2], didx0, isem0)

        pltpu.make_async_copy(w_hbm.at[sidx1], rows1, gsem1).wait()

        @pl.when(g + 2 < _ITERS)
        def _g2():
            pltpu.make_async_copy(ei_hbm.at[sbase], sidx0, isem0).wait()
            pltpu.make_async_copy(ei_hbm.at[dbase], didx0, isem0).wait()
            pltpu.async_copy(w_hbm.at[sidx0], rows0, gsem0)

        pltpu.sync_copy(rows1, acc.at[didx1], add=True)

        @pl.when(g + 3 < _ITERS)
        def _idx3():
            pltpu.async_copy(ei_hbm.at[sbase + g + 3], sidx1, isem1)
            pltpu.async_copy(ei_hbm.at[dbase + g + 3], didx1, isem1)

    plsc.subcore_barrier()

    doff = pl.multiple_of(s * _RPT8, 8)
    pltpu.sync_copy(acc.at[pl.ds(doff, _RPT8)],
                    out_hbm.at[c, pl.ds(doff, _RPT8)])

    @pl.when(s == _NS - 1)
    def _drain_tail():
        pltpu.sync_copy(acc.at[pl.ds(_NS * _RPT8, _TAIL)],
                        out_hbm.at[c, pl.ds(_NS * _RPT8, _TAIL)])


@functools.partial(
    pl.kernel,
    mesh=plsc.VectorSubcoreMesh(core_axis_name="c", subcore_axis_name="s"),
    out_type=jax.ShapeDtypeStruct((_NC, _N, _W), jnp.float32),
    scratch_types=[
        pltpu.VMEM((_C,), jnp.int32),
        pltpu.VMEM((_C,), jnp.int32),
        pltpu.VMEM((_C,), jnp.int32),
        pltpu.VMEM((_C,), jnp.int32),
        pltpu.VMEM((_C, _W), jnp.float32),
        pltpu.VMEM((_C, _W), jnp.float32),
        pltpu.VMEM_SHARED((_N, _W), jnp.float32),
        pltpu.SemaphoreType.DMA,
        pltpu.SemaphoreType.DMA,
        pltpu.SemaphoreType.DMA,
        pltpu.SemaphoreType.DMA,
    ],
    compiler_params=pltpu.CompilerParams(use_tc_tiling_on_sc=False),
)
def _edge_pass(w_hbm, ei_hbm, out_hbm,
               sidx0, sidx1, didx0, didx1, rows0, rows1, acc,
               gsem0, gsem1, isem0, isem1):
    _edge_body(w_hbm, ei_hbm, out_hbm,
               sidx0, sidx1, didx0, didx1, rows0, rows1, acc,
               gsem0, gsem1, isem0, isem1)


def _final_body(acc_ref, gselt_ref, out_ref):
    tot = acc_ref[0] + acc_ref[1]
    num = tot[:, 0:_HD]
    den = tot[:, _HD:_HD + _H]
    denb = jnp.dot(den, gselt_ref[...], precision=lax.Precision.HIGHEST)
    out_ref[...] = jnp.where(denb > 0, num / denb, 0.0)


def _final(acc, gselt):
    return pl.pallas_call(
        _final_body,
        out_shape=jax.ShapeDtypeStruct((_N, _HD), jnp.float32),
    )(acc, gselt)


def kernel(h_src, h_dst, attn_l, edge_index):
    del h_dst
    attn_flat = attn_l.reshape(1, _HD).astype(jnp.float32)
    gsel = (jnp.arange(_HD)[:, None] // _D == jnp.arange(_H)[None, :]
            ).astype(jnp.float32)
    gselt = gsel.T
    w = _prep(h_src, attn_flat, gsel, gselt)
    ei = edge_index.reshape(2 * _E // _C, _C)
    acc = _edge_pass(w, ei)
    return _final(acc, gselt)
